# baseline (speedup 1.0000x reference)
import functools

import jax
import jax.numpy as jnp
from jax.experimental import pallas as pl
from jax.experimental.pallas import tpu as pltpu


def _se_body(x, w1, w2, inv_hw):
    pooled = jnp.sum(x, axis=-1, dtype=jnp.float32)
    h = jnp.maximum(
        jax.lax.dot_general(pooled, w1, (((1,), (1,)), ((), ())),
                            preferred_element_type=jnp.float32) * inv_hw,
        0.0)
    s = jax.nn.sigmoid(
        jax.lax.dot_general(h, w2, (((1,), (1,)), ((), ())),
                            preferred_element_type=jnp.float32))
    return x * s[:, :, None]


def _se_kernel(x_ref, w1_ref, w2_ref, o_ref, *, inv_hw, nb, tail):
    w1 = w1_ref[...]
    w2 = w2_ref[...]
    i = pl.program_id(0)

    @pl.when(i < nb - 1)
    def _full():
        o_ref[...] = _se_body(x_ref[...], w1, w2, inv_hw)

    @pl.when(i == nb - 1)
    def _last():
        o_ref[:tail] = _se_body(x_ref[:tail], w1, w2, inv_hw)


def _se_block(x, w1, w2, bt):
    B, C, HW = x.shape
    nb = -(-B // bt)
    tail = B - (nb - 1) * bt
    itemsize = jnp.dtype(x.dtype).itemsize
    cr = int(w1.shape[0])
    cost = pl.CostEstimate(
        flops=2 * B * C * HW + 4 * B * C * cr,
        transcendentals=B * C,
        bytes_accessed=2 * B * C * HW * itemsize,
    )
    return pl.pallas_call(
        functools.partial(_se_kernel, inv_hw=1.0 / float(HW), nb=nb,
                          tail=tail),
        out_shape=jax.ShapeDtypeStruct((B, C, HW), x.dtype),
        grid=(nb,),
        in_specs=[
            pl.BlockSpec((bt, C, HW), lambda i: (i, 0, 0)),
            pl.BlockSpec(w1.shape, lambda i: (0, 0)),
            pl.BlockSpec(w2.shape, lambda i: (0, 0)),
        ],
        out_specs=pl.BlockSpec((bt, C, HW), lambda i: (i, 0, 0)),
        compiler_params=pltpu.CompilerParams(
            dimension_semantics=("parallel",),
            vmem_limit_bytes=100 * 1024 * 1024,
        ),
        cost_estimate=cost,
    )(x, w1, w2)


def kernel(x, w1, w2):
    B, C, H, W = x.shape
    xf = x.reshape(B, C, H * W)
    bt = 23 if B > 23 else B
    out = _se_block(xf, w1, w2, bt)
    return out.reshape(B, C, H, W)


# speedup vs baseline: 1.0008x; 1.0008x over previous
import functools

import jax
import jax.numpy as jnp
from jax.experimental import pallas as pl
from jax.experimental.pallas import tpu as pltpu


def _se_body(x, w1, w2, inv_hw):
    pooled = jnp.sum(x, axis=-1, dtype=jnp.float32)
    h = jnp.maximum(
        jax.lax.dot_general(pooled, w1, (((1,), (1,)), ((), ())),
                            preferred_element_type=jnp.float32) * inv_hw,
        0.0)
    s = jax.nn.sigmoid(
        jax.lax.dot_general(h, w2, (((1,), (1,)), ((), ())),
                            preferred_element_type=jnp.float32))
    return x * s[:, :, None]


def _se_kernel(x_ref, w1_ref, w2_ref, o_ref, *, inv_hw, nb, tail):
    w1 = w1_ref[...]
    w2 = w2_ref[...]
    i = pl.program_id(0)

    @pl.when(i < nb - 1)
    def _full():
        o_ref[...] = _se_body(x_ref[...], w1, w2, inv_hw)

    @pl.when(i == nb - 1)
    def _last():
        o_ref[:tail] = _se_body(x_ref[:tail], w1, w2, inv_hw)


def _se_block(x, w1, w2, bt):
    B, C, HW = x.shape
    nb = -(-B // bt)
    tail = B - (nb - 1) * bt
    itemsize = jnp.dtype(x.dtype).itemsize
    cr = int(w1.shape[0])
    cost = pl.CostEstimate(
        flops=2 * B * C * HW + 4 * B * C * cr,
        transcendentals=B * C,
        bytes_accessed=2 * B * C * HW * itemsize,
    )
    return pl.pallas_call(
        functools.partial(_se_kernel, inv_hw=1.0 / float(HW), nb=nb,
                          tail=tail),
        out_shape=jax.ShapeDtypeStruct((B, C, HW), x.dtype),
        grid=(nb,),
        in_specs=[
            pl.BlockSpec((bt, C, HW), lambda i: (i, 0, 0)),
            pl.BlockSpec(w1.shape, lambda i: (0, 0)),
            pl.BlockSpec(w2.shape, lambda i: (0, 0)),
        ],
        out_specs=pl.BlockSpec((bt, C, HW), lambda i: (i, 0, 0)),
        compiler_params=pltpu.CompilerParams(
            dimension_semantics=("parallel",),
            vmem_limit_bytes=100 * 1024 * 1024,
        ),
        cost_estimate=cost,
    )(x, w1, w2)


def kernel(x, w1, w2):
    B, C, H, W = x.shape
    xf = x.reshape(B, C, H * W)
    bt = 12 if B > 12 else B
    out = _se_block(xf, w1, w2, bt)
    return out.reshape(B, C, H, W)


# speedup vs baseline: 1.0036x; 1.0028x over previous
import functools

import jax
import jax.numpy as jnp
from jax.experimental import pallas as pl
from jax.experimental.pallas import tpu as pltpu


def _se_body(x, w1t, w2t, inv_hw):
    pooled = jnp.sum(x, axis=-1, dtype=jnp.float32)
    h = jnp.maximum(
        jnp.dot(pooled, w1t, preferred_element_type=jnp.float32) * inv_hw,
        0.0)
    s = jax.nn.sigmoid(
        jnp.dot(h, w2t, preferred_element_type=jnp.float32))
    return x * s[:, :, None]


def _se_kernel(x_ref, w1_ref, w2_ref, o_ref, *, inv_hw, nb, tail):
    w1t = w1_ref[...].T
    w2t = w2_ref[...].T
    o_ref[...] = _se_body(x_ref[...], w1t, w2t, inv_hw)


def _se_block(x, w1, w2, bt):
    B, C, HW = x.shape
    nb = -(-B // bt)
    tail = B - (nb - 1) * bt
    itemsize = jnp.dtype(x.dtype).itemsize
    cr = int(w1.shape[0])
    cost = pl.CostEstimate(
        flops=2 * B * C * HW + 4 * B * C * cr,
        transcendentals=B * C,
        bytes_accessed=2 * B * C * HW * itemsize,
    )
    return pl.pallas_call(
        functools.partial(_se_kernel, inv_hw=1.0 / float(HW), nb=nb,
                          tail=tail),
        out_shape=jax.ShapeDtypeStruct((B, C, HW), x.dtype),
        grid=(nb,),
        in_specs=[
            pl.BlockSpec((bt, C, HW), lambda i: (i, 0, 0)),
            pl.BlockSpec(w1.shape, lambda i: (0, 0)),
            pl.BlockSpec(w2.shape, lambda i: (0, 0)),
        ],
        out_specs=pl.BlockSpec((bt, C, HW), lambda i: (i, 0, 0)),
        compiler_params=pltpu.CompilerParams(
            dimension_semantics=("parallel",),
            vmem_limit_bytes=100 * 1024 * 1024,
        ),
        cost_estimate=cost,
    )(x, w1, w2)


def kernel(x, w1, w2):
    B, C, H, W = x.shape
    xf = x.reshape(B, C, H * W)
    bt = 23 if B > 23 else B
    out = _se_block(xf, w1, w2, bt)
    return out.reshape(B, C, H, W)


# speedup vs baseline: 1.0046x; 1.0010x over previous
import functools

import jax
import jax.numpy as jnp
from jax.experimental import pallas as pl
from jax.experimental.pallas import tpu as pltpu


def _se_kernel(x_ref, w1_ref, w2_ref, o_ref, *, inv_hw):
    w1t = w1_ref[...].T
    w2t = w2_ref[...].T
    pooled = jnp.sum(x_ref[...], axis=-1, dtype=jnp.float32) * inv_hw
    h = jnp.maximum(
        jnp.dot(pooled, w1t, preferred_element_type=jnp.float32), 0.0)
    s = jax.nn.sigmoid(
        jnp.dot(h, w2t, preferred_element_type=jnp.float32))
    gate = s[:, :, None].astype(o_ref.dtype)
    o_ref[...] = x_ref[...].astype(o_ref.dtype) * gate


def _se_block(x, w1, w2, bt):
    B, C, HW = x.shape
    nb = -(-B // bt)
    itemsize = jnp.dtype(x.dtype).itemsize
    cr = int(w1.shape[0])
    cost = pl.CostEstimate(
        flops=2 * B * C * HW + 4 * B * C * cr,
        transcendentals=B * C,
        bytes_accessed=2 * B * C * HW * itemsize
        + 2 * (w1.size + w2.size) * jnp.dtype(w1.dtype).itemsize,
    )
    return pl.pallas_call(
        functools.partial(_se_kernel, inv_hw=1.0 / float(HW)),
        out_shape=jax.ShapeDtypeStruct((B, C, HW), x.dtype),
        grid_spec=pltpu.PrefetchScalarGridSpec(
            num_scalar_prefetch=0,
            grid=(nb,),
            in_specs=[
                pl.BlockSpec((bt, C, HW), lambda i: (i, 0, 0)),
                pl.BlockSpec(w1.shape, lambda i: (0, 0)),
                pl.BlockSpec(w2.shape, lambda i: (0, 0)),
            ],
            out_specs=pl.BlockSpec((bt, C, HW), lambda i: (i, 0, 0)),
        ),
        compiler_params=pltpu.CompilerParams(
            dimension_semantics=("parallel",),
            vmem_limit_bytes=36 * 1024 * 1024,
        ),
        cost_estimate=cost,
    )(x, w1, w2)


def kernel(x, w1, w2):
    B, C, H, W = x.shape
    xf = x.reshape(B, C, H * W)
    bt = 12 if B > 12 else B
    out = _se_block(xf, w1, w2, bt)
    return out.reshape(B, C, H, W)


# speedup vs baseline: 1.0063x; 1.0017x over previous
import functools

import jax
import jax.numpy as jnp
from jax.experimental import pallas as pl
from jax.experimental.pallas import tpu as pltpu


def _se_kernel(x_ref, w1_ref, w2_ref, o_ref, w1t_ref, w2t_ref, *, inv_hw):
    j = pl.program_id(1)

    @pl.when(j == 0)
    def _prep():
        w1t_ref[...] = w1_ref[...].T
        w2t_ref[...] = w2_ref[...].T

    pooled = jnp.sum(x_ref[...], axis=-1, dtype=jnp.float32) * inv_hw
    h = jnp.maximum(
        jnp.dot(pooled, w1t_ref[...], preferred_element_type=jnp.float32),
        0.0)
    s = jax.nn.sigmoid(
        jnp.dot(h, w2t_ref[...], preferred_element_type=jnp.float32))
    gate = s[:, :, None].astype(o_ref.dtype)
    o_ref[...] = x_ref[...].astype(o_ref.dtype) * gate


def _se_block(x, w1, w2, bt):
    B, C, HW = x.shape
    nb = -(-B // bt)
    nb2 = nb // 2
    itemsize = jnp.dtype(x.dtype).itemsize
    cr = int(w1.shape[0])
    cost = pl.CostEstimate(
        flops=2 * B * C * HW + 4 * B * C * cr,
        transcendentals=B * C,
        bytes_accessed=2 * B * C * HW * itemsize
        + 2 * (w1.size + w2.size) * jnp.dtype(w1.dtype).itemsize,
    )
    return pl.pallas_call(
        functools.partial(_se_kernel, inv_hw=1.0 / float(HW)),
        out_shape=jax.ShapeDtypeStruct((B, C, HW), x.dtype),
        grid_spec=pltpu.PrefetchScalarGridSpec(
            num_scalar_prefetch=0,
            grid=(2, nb2),
            in_specs=[
                pl.BlockSpec((bt, C, HW), lambda i, j: (i * nb2 + j, 0, 0)),
                pl.BlockSpec(w1.shape, lambda i, j: (0, 0)),
                pl.BlockSpec(w2.shape, lambda i, j: (0, 0)),
            ],
            out_specs=pl.BlockSpec((bt, C, HW),
                                   lambda i, j: (i * nb2 + j, 0, 0)),
            scratch_shapes=[
                pltpu.VMEM((C, cr), jnp.float32),
                pltpu.VMEM((cr, C), jnp.float32),
            ],
        ),
        compiler_params=pltpu.CompilerParams(
            dimension_semantics=("parallel", "arbitrary"),
            vmem_limit_bytes=36 * 1024 * 1024,
        ),
        cost_estimate=cost,
    )(x, w1, w2)


def kernel(x, w1, w2):
    B, C, H, W = x.shape
    xf = x.reshape(B, C, H * W)
    bt = 12 if B > 12 else B
    out = _se_block(xf, w1, w2, bt)
    return out.reshape(B, C, H, W)


# speedup vs baseline: 1.0089x; 1.0026x over previous
import functools

import jax
import jax.numpy as jnp
from jax.experimental import pallas as pl
from jax.experimental.pallas import tpu as pltpu


def _se_kernel(x_ref, w1_ref, w2_ref, o_ref, w1t_ref, w2t_ref, *, inv_hw):
    j = pl.program_id(1)

    @pl.when(j == 0)
    def _prep():
        w1t_ref[...] = w1_ref[...].T
        w2t_ref[...] = w2_ref[...].T

    pooled = jnp.sum(x_ref[...], axis=-1, dtype=jnp.float32) * inv_hw
    h = jnp.maximum(
        jnp.dot(pooled, w1t_ref[...], preferred_element_type=jnp.float32),
        0.0)
    s = jax.nn.sigmoid(
        jnp.dot(h, w2t_ref[...], preferred_element_type=jnp.float32))
    gate = s[:, :, None].astype(o_ref.dtype)
    o_ref[...] = x_ref[...].astype(o_ref.dtype) * gate


def _se_block(x, w1, w2, bt):
    B, C, HW = x.shape
    nb = -(-B // bt)
    nb2 = nb // 2
    itemsize = jnp.dtype(x.dtype).itemsize
    cr = int(w1.shape[0])
    cost = pl.CostEstimate(
        flops=2 * B * C * HW + 4 * B * C * cr,
        transcendentals=B * C,
        bytes_accessed=2 * B * C * HW * itemsize
        + 2 * (w1.size + w2.size) * jnp.dtype(w1.dtype).itemsize,
    )
    return pl.pallas_call(
        functools.partial(_se_kernel, inv_hw=1.0 / float(HW)),
        out_shape=jax.ShapeDtypeStruct((B, C, HW), x.dtype),
        grid_spec=pltpu.PrefetchScalarGridSpec(
            num_scalar_prefetch=0,
            grid=(2, nb2),
            in_specs=[
                pl.BlockSpec((bt, C, HW), lambda i, j: (i * nb2 + j, 0, 0)),
                pl.BlockSpec(w1.shape, lambda i, j: (0, 0)),
                pl.BlockSpec(w2.shape, lambda i, j: (0, 0)),
            ],
            out_specs=pl.BlockSpec((bt, C, HW),
                                   lambda i, j: (i * nb2 + j, 0, 0)),
            scratch_shapes=[
                pltpu.VMEM((C, cr), jnp.float32),
                pltpu.VMEM((cr, C), jnp.float32),
            ],
        ),
        compiler_params=pltpu.CompilerParams(
            dimension_semantics=("parallel", "arbitrary"),
            vmem_limit_bytes=52 * 1024 * 1024,
        ),
        cost_estimate=cost,
    )(x, w1, w2)


def kernel(x, w1, w2):
    B, C, H, W = x.shape
    xf = x.reshape(B, C, H * W)
    bt = 23 if B > 23 else B
    out = _se_block(xf, w1, w2, bt)
    return out.reshape(B, C, H, W)
